# baseline (speedup 1.0000x reference)
import functools

import jax
import jax.numpy as jnp
from jax import lax
from jax.experimental import pallas as pl
from jax.experimental.pallas import tpu as pltpu
from jax.experimental.pallas import tpu_sc as plsc

NUM_EMB = 1_000_000
DIM = 64
BATCH = 16384
HIST = 50
B_TOTAL = BATCH * HIST
NUM_CORES = 2
NUM_SUBCORES = 16
NW = NUM_CORES * NUM_SUBCORES
NBLK = BATCH // 128
BLK_PER_W = NBLK // NW
NH = 5
HC = HIST // NH
CHUNK = 128 * NH
N_CHUNKS = BLK_PER_W * HC
B_PER_W = N_CHUNKS * CHUNK
TILES = NH * 8

_mesh = plsc.VectorSubcoreMesh(core_axis_name="c", subcore_axis_name="s")


@functools.partial(
    pl.kernel,
    mesh=_mesh,
    out_type=jax.ShapeDtypeStruct((HIST, 8, NBLK, 8, 128), jnp.float32),
    scratch_types=[
        pltpu.VMEM((2, CHUNK), jnp.int32),
        pltpu.VMEM((2, CHUNK, DIM), jnp.float32),
        pltpu.VMEM((8, 128), jnp.float32),
        pltpu.VMEM((8, 128), jnp.float32),
        pltpu.VMEM((2, 16, 64), jnp.float32),
        pltpu.SemaphoreType.DMA,
        pltpu.SemaphoreType.DMA,
        pltpu.SemaphoreType.DMA,
        pltpu.SemaphoreType.DMA,
    ],
    compiler_params=pltpu.CompilerParams(
        use_tc_tiling_on_sc=False, needs_layout_passes=False
    ),
)
def _gather_kernel(idx_hbm, table_hbm, out_hbm, idx2, rows2, tileA, tileB,
                   dump, g0, g1, wA, wB):
    wid = lax.axis_index("s") * NUM_CORES + lax.axis_index("c")
    base = wid * B_PER_W
    blk0 = wid * BLK_PER_W
    iota = lax.iota(jnp.int32, 16)
    gsem = (g0, g1)
    tbufs = ((tileA, wA), (tileB, wB))

    def fire_gather(c, par):
        pltpu.sync_copy(idx_hbm.at[pl.ds(base + c * CHUNK, CHUNK)], idx2.at[par])
        pltpu.async_copy(table_hbm.at[idx2.at[par]], rows2.at[par], gsem[par])

    def drain_gather(par):
        pltpu.make_async_copy(
            table_hbm.at[pl.ds(0, CHUNK)], rows2.at[par], gsem[par]
        ).wait()

    def drain_tile(tile_v, sem):
        pltpu.make_async_copy(tile_v, out_hbm.at[0, 0, 0], sem).wait()

    def fill_tiles(c, par):
        blk = blk0 + c // HC
        h0 = (c % HC) * NH
        rows = rows2.at[par]

        def tile_pair(tt, carry):
            for delta, (tile_v, sem) in ((0, tbufs[0]), (1, tbufs[1])):
                t = 2 * tt + delta
                h1 = t // 8
                g = t - 8 * h1
                drain_tile(tile_v, sem)
                rowbase = iota + h1 * 128
                rowvs = [rowbase + 16 * k for k in range(8)]
                colvs = [jnp.full((16,), 8 * g + dd, jnp.int32)
                         for dd in range(8)]
                for dd in range(0, 8, 4):
                    vecs = [plsc.load_gather(rows, [rowvs[k], colvs[dd + j]])
                            for j in range(4) for k in range(8)]
                    for j in range(4):
                        for k in range(8):
                            tile_v[dd + j, pl.ds(16 * k, 16)] = vecs[j * 8 + k]
                pltpu.async_copy(tile_v, out_hbm.at[h0 + h1, g, blk], sem)
            return carry

        lax.fori_loop(0, TILES // 2, tile_pair, 0)

    pltpu.async_copy(table_hbm.at[pl.ds(0, 16)], dump.at[0], wA)
    pltpu.async_copy(table_hbm.at[pl.ds(0, 16)], dump.at[1], wB)

    fire_gather(0, 0)
    fire_gather(1, 1)
    drain_gather(0)
    fill_tiles(0, 0)

    def pair_body(p, carry):
        c1 = 1 + 2 * p
        fire_gather(c1 + 1, 0)
        drain_gather(1)
        fill_tiles(c1, 1)
        fire_gather(c1 + 2, 1)
        drain_gather(0)
        fill_tiles(c1 + 1, 0)
        return carry

    lax.fori_loop(0, (N_CHUNKS - 2) // 2, pair_body, 0)

    drain_gather(1)
    fill_tiles(N_CHUNKS - 1, 1)

    drain_tile(tileA, wA)
    drain_tile(tileB, wB)


def kernel(token_ids, emb):
    ids_r = (
        token_ids.reshape(NBLK, 128, HIST)
        .transpose(0, 2, 1)
        .reshape(-1)
        .astype(jnp.int32)
    )
    out5 = _gather_kernel(ids_r, emb)
    return out5.transpose(2, 4, 0, 1, 3).reshape(BATCH, HIST, DIM)


# speedup vs baseline: 1.2742x; 1.2742x over previous
import functools

import jax
import jax.numpy as jnp
from jax import lax
from jax.experimental import pallas as pl
from jax.experimental.pallas import tpu as pltpu
from jax.experimental.pallas import tpu_sc as plsc

NUM_EMB = 1_000_000
DIM = 64
BATCH = 16384
HIST = 50
B_TOTAL = BATCH * HIST
NUM_CORES = 2
NUM_SUBCORES = 16
NW = NUM_CORES * NUM_SUBCORES
B_PER_W = B_TOTAL // NW
CHUNK = 800
N_CHUNKS = B_PER_W // CHUNK
N_PAIRS = N_CHUNKS // 2

_mesh = plsc.VectorSubcoreMesh(core_axis_name="c", subcore_axis_name="s")


@functools.partial(
    pl.kernel,
    mesh=_mesh,
    out_type=jax.ShapeDtypeStruct((B_TOTAL, DIM), jnp.float32),
    scratch_types=[
        pltpu.VMEM((2, CHUNK), jnp.int32),
        pltpu.VMEM((2, CHUNK, DIM), jnp.float32),
        pltpu.SemaphoreType.DMA,
        pltpu.SemaphoreType.DMA,
        pltpu.SemaphoreType.DMA,
        pltpu.SemaphoreType.DMA,
    ],
    compiler_params=pltpu.CompilerParams(use_tc_tiling_on_sc=False),
)
def _gather_kernel(idx_hbm, table_hbm, out_hbm, idx2, rows2, gA, gB, wA, wB):
    wid = lax.axis_index("s") * NUM_CORES + lax.axis_index("c")
    base = wid * B_PER_W
    idxA, idxB = idx2.at[0], idx2.at[1]
    rowsA, rowsB = rows2.at[0], rows2.at[1]

    def fire_gather(idx_ref, rows_ref, off, sem):
        pltpu.sync_copy(idx_hbm.at[pl.ds(off, CHUNK)], idx_ref)
        return pltpu.async_copy(table_hbm.at[idx_ref], rows_ref, sem)

    def fire_write(rows_ref, off, sem):
        return pltpu.async_copy(rows_ref, out_hbm.at[pl.ds(off, CHUNK)], sem)

    def drain_gather(rows_ref, sem):
        pltpu.make_async_copy(table_hbm.at[pl.ds(0, CHUNK)], rows_ref, sem).wait()

    def drain_write(rows_ref, sem):
        pltpu.make_async_copy(rows_ref, out_hbm.at[pl.ds(base, CHUNK)], sem).wait()

    dA = fire_gather(idxA, rowsA, base, gA)
    dB = fire_gather(idxB, rowsB, base + CHUNK, gB)
    dA.wait()
    fire_write(rowsA, base, wA)
    dB.wait()
    fire_write(rowsB, base + CHUNK, wB)
    drain_write(rowsA, wA)
    fire_gather(idxA, rowsA, base + 2 * CHUNK, gA)

    def body(g, carry):
        offA = base + (2 * g) * CHUNK
        offB = offA + CHUNK
        drain_write(rowsB, wB)
        dBg = fire_gather(idxB, rowsB, offB, gB)
        drain_gather(rowsA, gA)
        fire_write(rowsA, offA, wA)
        dBg.wait()
        fire_write(rowsB, offB, wB)
        drain_write(rowsA, wA)
        fire_gather(idxA, rowsA, offA + 2 * CHUNK, gA)
        return carry

    lax.fori_loop(1, N_PAIRS - 1, body, 0)

    offA = base + (N_CHUNKS - 2) * CHUNK
    offB = offA + CHUNK
    drain_write(rowsB, wB)
    dBl = fire_gather(idxB, rowsB, offB, gB)
    drain_gather(rowsA, gA)
    fire_write(rowsA, offA, wA)
    dBl.wait()
    fire_write(rowsB, offB, wB)
    drain_write(rowsA, wA)
    drain_write(rowsB, wB)


def kernel(token_ids, emb):
    flat_ids = token_ids.reshape(-1).astype(jnp.int32)
    out = _gather_kernel(flat_ids, emb)
    return out.reshape(BATCH, HIST, DIM)


# speedup vs baseline: 1.3404x; 1.0520x over previous
import functools

import jax
import jax.numpy as jnp
from jax import lax
from jax.experimental import pallas as pl
from jax.experimental.pallas import tpu as pltpu
from jax.experimental.pallas import tpu_sc as plsc

NUM_EMB = 1_000_000
DIM = 64
BATCH = 16384
HIST = 50
B_TOTAL = BATCH * HIST
NUM_CORES = 2
NUM_SUBCORES = 16
NW = NUM_CORES * NUM_SUBCORES
B_PER_W = B_TOTAL // NW
CHUNK = 800
N_CHUNKS = B_PER_W // CHUNK
N_PAIRS = N_CHUNKS // 2

_mesh = plsc.VectorSubcoreMesh(core_axis_name="c", subcore_axis_name="s")


@functools.partial(
    pl.kernel,
    mesh=_mesh,
    out_type=jax.ShapeDtypeStruct((B_TOTAL, DIM), jnp.float32),
    scratch_types=[
        pltpu.VMEM((2, CHUNK), jnp.int32),
        pltpu.VMEM((2, CHUNK, DIM), jnp.float32),
        pltpu.SemaphoreType.DMA,
        pltpu.SemaphoreType.DMA,
        pltpu.SemaphoreType.DMA,
        pltpu.SemaphoreType.DMA,
    ],
    compiler_params=pltpu.CompilerParams(use_tc_tiling_on_sc=False),
)
def _gather_kernel(idx_hbm, table_hbm, out_hbm, idx2, rows2, gA, gB, wA, wB):
    wid = lax.axis_index("s") * NUM_CORES + lax.axis_index("c")
    base = wid * B_PER_W
    idxA, idxB = idx2.at[0], idx2.at[1]
    rowsA, rowsB = rows2.at[0], rows2.at[1]

    def fire_gather(idx_ref, rows_ref, off, sem):
        pltpu.sync_copy(idx_hbm.at[pl.ds(off, CHUNK)], idx_ref)
        return pltpu.async_copy(table_hbm.at[idx_ref], rows_ref, sem)

    def fire_write(rows_ref, off, sem):
        return pltpu.async_copy(rows_ref, out_hbm.at[pl.ds(off, CHUNK)], sem)

    def drain_gather(rows_ref, sem):
        pltpu.make_async_copy(table_hbm.at[pl.ds(0, CHUNK)], rows_ref, sem).wait()

    def drain_write(rows_ref, sem):
        pltpu.make_async_copy(rows_ref, out_hbm.at[pl.ds(base, CHUNK)], sem).wait()

    dA = fire_gather(idxA, rowsA, base, gA)
    dB = fire_gather(idxB, rowsB, base + CHUNK, gB)
    dA.wait()
    fire_write(rowsA, base, wA)
    dB.wait()
    fire_write(rowsB, base + CHUNK, wB)
    drain_write(rowsA, wA)
    fire_gather(idxA, rowsA, base + 2 * CHUNK, gA)

    def body(g, carry):
        offA = base + (2 * g) * CHUNK
        offB = offA + CHUNK
        drain_write(rowsB, wB)
        dBg = fire_gather(idxB, rowsB, offB, gB)
        drain_gather(rowsA, gA)
        fire_write(rowsA, offA, wA)
        dBg.wait()
        fire_write(rowsB, offB, wB)
        drain_write(rowsA, wA)
        fire_gather(idxA, rowsA, offA + 2 * CHUNK, gA)
        return carry

    lax.fori_loop(1, N_PAIRS - 1, body, 0)

    offA = base + (N_CHUNKS - 2) * CHUNK
    offB = offA + CHUNK
    drain_write(rowsB, wB)
    dBl = fire_gather(idxB, rowsB, offB, gB)
    drain_gather(rowsA, gA)
    fire_write(rowsA, offA, wA)
    dBl.wait()
    fire_write(rowsB, offB, wB)
    drain_write(rowsA, wA)
    drain_write(rowsB, wB)


def kernel(token_ids, emb):
    flat_ids = (token_ids.reshape(-1) * 2).astype(jnp.int32)
    emb_p = jnp.pad(emb, ((0, 0), (0, DIM))).reshape(2 * NUM_EMB, DIM)
    out = _gather_kernel(flat_ids, emb_p)
    return out.reshape(BATCH, HIST, DIM)
